# baseline (speedup 1.0000x reference)
import functools

import jax
import jax.numpy as jnp
from jax import lax
from jax.experimental import pallas as pl
from jax.experimental.pallas import tpu as pltpu
from jax.experimental.pallas import tpu_sc as plsc

XMAX = 100.0
ALPHA = 0.75
NW = 32
CHUNK = 128
GROUP = 16


def _prep_body(x_ref, lw_ref, cf_ref):
    x = x_ref[...]
    lw = jnp.log(x)
    cf = jnp.minimum(jnp.exp(ALPHA * (lw - jnp.log(XMAX))), 1.0)
    lw_ref[...] = lw
    cf_ref[...] = cf


def _sc_body(ppw, E, i_hbm, j_hbm, lw_hbm, cf_hbm, w_hbm, wp_hbm, b_hbm,
             bp_hbm, out_hbm, idx_i, idx_j, wi_v, wj_v, bi_v, bj_v, lw_v,
             cf_v, acc_v, sem):
    nch = ppw // CHUNK
    ngrp = ppw // GROUP
    c = lax.axis_index("c")
    s = lax.axis_index("s")
    wid = s * 2 + c

    pltpu.sync_copy(i_hbm.at[wid], idx_i)
    pltpu.sync_copy(j_hbm.at[wid], idx_j)
    pltpu.sync_copy(lw_hbm.at[wid], lw_v)
    pltpu.sync_copy(cf_hbm.at[wid], cf_v)

    copies = []
    for ch in range(nch):
        r0 = ch * CHUNK
        copies.append(pltpu.async_copy(
            w_hbm.at[idx_i.at[ch]], wi_v.at[pl.ds(r0, CHUNK), :], sem))
        copies.append(pltpu.async_copy(
            wp_hbm.at[idx_j.at[ch]], wj_v.at[pl.ds(r0, CHUNK), :], sem))
        copies.append(pltpu.async_copy(
            b_hbm.at[idx_i.at[ch]], bi_v.at[ch], sem))
        copies.append(pltpu.async_copy(
            bp_hbm.at[idx_j.at[ch]], bj_v.at[ch], sem))
    for cp in copies:
        cp.wait()

    lanes = lax.iota(jnp.int32, GROUP)
    rot_idx = [(lanes + sh) % GROUP for sh in (8, 4, 2, 1)]

    def hsum(p):
        for idx in rot_idx:
            p = p + p.at[idx].get(mode="promise_in_bounds")
        return p

    def group_step(g, acc):
        base = g * GROUP
        dots = jnp.zeros((GROUP,), jnp.float32)
        for rr in range(GROUP):
            r = base + rr
            p = wi_v[r, pl.ds(0, 16)] * wj_v[r, pl.ds(0, 16)]
            for k in range(1, E // 16):
                p = p + wi_v[r, pl.ds(16 * k, 16)] * wj_v[r, pl.ds(16 * k, 16)]
            dots = jnp.where(lanes == rr, hsum(p), dots)
        ch = g // (CHUNK // GROUP)
        off = (g % (CHUNK // GROUP)) * GROUP
        bi = bi_v[ch, pl.ds(off, 16)]
        bj = bj_v[ch, pl.ds(off, 16)]
        lwg = lw_v[pl.ds(base, 16)]
        cfg = cf_v[pl.ds(base, 16)]
        err = dots + bi + bj - lwg
        return acc + cfg * err * err

    acc = lax.fori_loop(0, ngrp, group_step, jnp.zeros((16,), jnp.float32))
    acc_v[...] = acc
    pltpu.sync_copy(acc_v, out_hbm.at[wid])


def kernel(i, j, xij, W, W_, b, b_):
    B = i.shape[0]
    V, E = W.shape
    ppw = B // NW
    nch = ppw // CHUNK

    x2d = xij.reshape(B // 128, 128)
    lw2d, cf2d = pl.pallas_call(
        _prep_body,
        out_shape=(jax.ShapeDtypeStruct(x2d.shape, jnp.float32),
                   jax.ShapeDtypeStruct(x2d.shape, jnp.float32)),
    )(x2d)

    i3 = jnp.asarray(i, jnp.int32).reshape(NW, nch, CHUNK)
    j3 = jnp.asarray(j, jnp.int32).reshape(NW, nch, CHUNK)
    lw = lw2d.reshape(NW, ppw)
    cf = cf2d.reshape(NW, ppw)

    mesh = plsc.VectorSubcoreMesh(core_axis_name="c", subcore_axis_name="s")
    sc = functools.partial(
        pl.kernel,
        mesh=mesh,
        compiler_params=pltpu.CompilerParams(use_tc_tiling_on_sc=False),
        out_type=jax.ShapeDtypeStruct((NW, GROUP), jnp.float32),
        scratch_types=[
            pltpu.VMEM((nch, CHUNK), jnp.int32),
            pltpu.VMEM((nch, CHUNK), jnp.int32),
            pltpu.VMEM((ppw, E), jnp.float32),
            pltpu.VMEM((ppw, E), jnp.float32),
            pltpu.VMEM((nch, CHUNK), jnp.float32),
            pltpu.VMEM((nch, CHUNK), jnp.float32),
            pltpu.VMEM((ppw,), jnp.float32),
            pltpu.VMEM((ppw,), jnp.float32),
            pltpu.VMEM((GROUP,), jnp.float32),
            pltpu.SemaphoreType.DMA,
        ],
    )(functools.partial(_sc_body, ppw, E))

    partials = sc(i3, j3, lw, cf, W, W_, b, b_)
    return jnp.sum(partials) / B
